# baseline (speedup 1.0000x reference)
import functools

import jax
import jax.numpy as jnp
from jax.experimental import pallas as pl
from jax.experimental.pallas import tpu as pltpu
from jax.experimental.pallas import tpu_sc as plsc

D_EM = 64
K_RET = 32
CH = 128
G = 896
MP = G * CH
TM = 16384
NEG = -1.0e30
NEGBIG = -3.0e38


def _q_body(x_ref, y_ref, wqem_ref, bqem_ref, wqc_ref, bqc_ref, q_ref, qc_ref):
    x = x_ref[0]
    y = y_ref[0]
    w = wqem_ref[...]
    q = (jax.lax.dot_general(x, w[:, :1024], (((1,), (1,)), ((), ())),
                             preferred_element_type=jnp.float32)
         + jax.lax.dot_general(y, w[:, 1024:], (((1,), (1,)), ((), ())),
                               preferred_element_type=jnp.float32)
         + bqem_ref[...][None, :])
    q = q * jax.lax.rsqrt(jnp.sum(q * q, axis=1, keepdims=True) + 1e-12)
    q_ref[0] = q
    qc_ref[0] = (jax.lax.dot_general(x, wqc_ref[...], (((1,), (1,)), ((), ())),
                                     preferred_element_type=jnp.float32)
                 + bqc_ref[...][None, :])


def _scores_body(q_ref, k_ref, s_ref, scores_ref, cmax_ref):
    m = pl.program_id(1)
    s = jax.lax.dot_general(q_ref[0], k_ref[0], (((1,), (0,)), ((), ())),
                            preferred_element_type=jnp.float32)
    col = m * TM + jax.lax.broadcasted_iota(jnp.int32, s.shape, 1)
    active = (s_ref[0, 0][None, :] > 0.0) & (col < 100000)
    s = jnp.where(active, s, NEG).reshape(s.shape[0], TM // CH, CH)
    scores_ref[0] = s
    cmax_ref[0] = jnp.max(s, axis=2)


def _topk_extract(c, pay, k):
    iota_k = jax.lax.broadcasted_iota(jnp.int32, (c.shape[0], k), 1)

    def body(i, carry):
        c, m, vals, idx = carry
        j = jnp.min(jnp.where(c >= m, pay, jnp.int32(2**31 - 1)),
                    axis=1, keepdims=True)
        c = jnp.where(pay == j, NEGBIG, c)
        sel = iota_k == i
        vals = jnp.where(sel, m, vals)
        idx = jnp.where(sel, j, idx)
        return c, jnp.max(c, axis=1, keepdims=True), vals, idx

    vals0 = jnp.full((c.shape[0], k), NEGBIG, jnp.float32)
    idx0 = jnp.zeros((c.shape[0], k), jnp.int32)
    m0 = jnp.max(c, axis=1, keepdims=True)
    _, _, vals, idx = jax.lax.fori_loop(0, k, body, (c, m0, vals0, idx0))
    return vals, idx


def _chunk_topk_body(cmax_ref, cidx_ref):
    c = cmax_ref[0]
    _, idx = _topk_extract(c, jax.lax.broadcasted_iota(jnp.int32, c.shape, 1),
                           K_RET)
    cidx_ref[0] = idx


def _cand_topk_body(cand_ref, cidx_ref, vals_ref, jidx_ref):
    c0 = cand_ref[0]
    p = c0.shape[0]
    gidx = (cidx_ref[0][:, :, None] * CH
            + jax.lax.broadcasted_iota(jnp.int32, (p, K_RET, CH), 2))
    iota_k = jax.lax.broadcasted_iota(jnp.int32, (p, K_RET), 1)

    def body(i, carry):
        c, m, vals, idx = carry
        j = jnp.min(jnp.where(c >= m[:, :, None], gidx,
                              jnp.int32(2**31 - 1)),
                    axis=(1, 2), keepdims=True)[:, :, 0]
        c = jnp.where(gidx == j[:, :, None], NEGBIG, c)
        sel = iota_k == i
        vals = jnp.where(sel, m, vals)
        idx = jnp.where(sel, j, idx)
        m = jnp.max(c, axis=(1, 2), keepdims=True)[:, :, 0]
        return c, m, vals, idx

    vals0 = jnp.full((p, K_RET), NEGBIG, jnp.float32)
    idx0 = jnp.zeros((p, K_RET), jnp.int32)
    m0 = jnp.max(c0, axis=(1, 2), keepdims=True)[:, :, 0]
    _, _, vals, idx = jax.lax.fori_loop(0, K_RET, body, (c0, m0, vals0, idx0))
    vals_ref[0] = vals
    jidx_ref[0] = idx


TV = 8192
GV = 13
MV2 = GV * TV // 2
LOG2TV = 13


def _repack_v_body(vt_ref, out_ref):
    v = jnp.swapaxes(vt_ref[0], 0, 1)
    out_ref[0] = jnp.concatenate([v[:TV // 2], v[TV // 2:]], axis=1)


def _combine_body(qc_ref, v_ref, vals_ref, jid_ref, wo_ref, bo_ref, y_ref):
    qc = qc_ref[0]
    v2 = v_ref[0]
    vals = vals_ref[0]
    parity = jnp.bitwise_and(
        jax.lax.shift_right_logical(jid_ref[0], LOG2TV - 1), 1)[:, :, None]
    v = jnp.where(parity == 1, v2[:, :, D_EM:], v2[:, :, :D_EM])
    t = jnp.sum(qc[:, None, :] * v, axis=2) * (D_EM ** -0.5)
    masked = vals < -1.0e29
    logits = jnp.where(masked, NEGBIG, t + vals)
    mx = jnp.max(logits, axis=1, keepdims=True)
    e = jnp.where(masked, 0.0, jnp.exp(logits - mx))
    ssum = jnp.sum(e, axis=1, keepdims=True)
    attn = e / jnp.maximum(ssum, 1e-30)
    out = jnp.sum(attn[:, :, None] * v, axis=1)
    y_ref[0] = (jax.lax.dot_general(out, wo_ref[...], (((1,), (1,)), ((), ())),
                                    preferred_element_type=jnp.float32)
                + bo_ref[...][None, :])


def _gather_chunks_sc(scores_flat, cid_flat):
    mesh = plsc.VectorSubcoreMesh(core_axis_name="c", subcore_axis_name="s")

    @functools.partial(
        pl.kernel, mesh=mesh,
        out_type=jax.ShapeDtypeStruct((16384, CH), jnp.float32),
        scratch_types=[
            pltpu.VMEM((512,), jnp.int32),
            pltpu.VMEM((4, 128), jnp.int32),
            pltpu.VMEM((512, CH), jnp.float32),
            pltpu.SemaphoreType.DMA,
        ],
    )
    def k(tab, cid, out, cid_v, idx_v, rows_v, sem):
        wid = jax.lax.axis_index("s") * 2 + jax.lax.axis_index("c")
        g0 = wid * 16
        pltpu.sync_copy(cid.at[pl.ds(g0 * K_RET, 512)], cid_v)
        for i in range(32):
            v = cid_v[pl.ds(i * 16, 16)]
            idx_v[i // 8, pl.ds((i % 8) * 16, 16)] = v + (g0 + i // 2) * G
        copies = [
            pltpu.async_copy(tab.at[idx_v.at[j]],
                             rows_v.at[pl.ds(j * 128, 128)], sem)
            for j in range(4)
        ]
        for c in copies:
            c.wait()
        pltpu.sync_copy(rows_v, out.at[pl.ds(g0 * K_RET, 512)])

    return k(scores_flat, cid_flat)


def _gather_v_sc(v_pairs, jid_flat):
    mesh = plsc.VectorSubcoreMesh(core_axis_name="c", subcore_axis_name="s")

    @functools.partial(
        pl.kernel, mesh=mesh,
        out_type=jax.ShapeDtypeStruct((16384, 2 * D_EM), jnp.float32),
        scratch_types=[
            pltpu.VMEM((512,), jnp.int32),
            pltpu.VMEM((4, 128), jnp.int32),
            pltpu.VMEM((512, 2 * D_EM), jnp.float32),
            pltpu.SemaphoreType.DMA,
        ],
    )
    def k(tab, jid, out, jid_v, idx_v, rows_v, sem):
        wid = jax.lax.axis_index("s") * 2 + jax.lax.axis_index("c")
        g0 = wid * 16
        pltpu.sync_copy(jid.at[pl.ds(g0 * K_RET, 512)], jid_v)
        for i in range(32):
            jv = jnp.minimum(jid_v[pl.ds(i * 16, 16)], 100000 - 1)
            b = jax.lax.div(g0 + i // 2, 128)
            t = jax.lax.shift_right_logical(jv, LOG2TV)
            rr = jnp.bitwise_and(jv, TV // 2 - 1)
            idx_v[i // 8, pl.ds((i % 8) * 16, 16)] = (
                b * MV2 + t * (TV // 2) + rr)
        copies = [
            pltpu.async_copy(tab.at[idx_v.at[j]],
                             rows_v.at[pl.ds(j * 128, 128)], sem)
            for j in range(4)
        ]
        for c in copies:
            c.wait()
        pltpu.sync_copy(rows_v, out.at[pl.ds(g0 * K_RET, 512)])

    return k(v_pairs, jid_flat)


def kernel(x_all, y_wm_all, em_K, em_V, em_S, W_q_em, b_q_em, W_q_cross,
           b_q_cross, W_o_cross, b_o_cross):
    B, P, D = x_all.shape
    M = em_K.shape[1]

    q, q_cross = pl.pallas_call(
        _q_body,
        out_shape=(jax.ShapeDtypeStruct((B, P, D_EM), jnp.float32),
                   jax.ShapeDtypeStruct((B, P, D_EM), jnp.float32)),
        grid=(B,),
        in_specs=[
            pl.BlockSpec((1, P, D), lambda b: (b, 0, 0)),
            pl.BlockSpec((1, P, D), lambda b: (b, 0, 0)),
            pl.BlockSpec((D_EM, 2 * D), lambda b: (0, 0)),
            pl.BlockSpec((D_EM,), lambda b: (0,)),
            pl.BlockSpec((D_EM, D), lambda b: (0, 0)),
            pl.BlockSpec((D_EM,), lambda b: (0,)),
        ],
        out_specs=(pl.BlockSpec((1, P, D_EM), lambda b: (b, 0, 0)),
                   pl.BlockSpec((1, P, D_EM), lambda b: (b, 0, 0))),
    )(x_all, y_wm_all, W_q_em, b_q_em, W_q_cross, b_q_cross)

    em_Sp = jnp.pad(em_S, ((0, 0), (0, MP - M))).reshape(B * (MP // TM), 1, TM)

    scores, cmax = pl.pallas_call(
        _scores_body,
        out_shape=(jax.ShapeDtypeStruct((B, P, G, CH), jnp.float32),
                   jax.ShapeDtypeStruct((B, P, G), jnp.float32)),
        grid=(B, MP // TM),
        in_specs=[
            pl.BlockSpec((1, P, D_EM), lambda b, m: (b, 0, 0)),
            pl.BlockSpec((1, D_EM, TM), lambda b, m: (b, 0, m)),
            pl.BlockSpec((1, 1, TM), lambda b, m: (b * (MP // TM) + m, 0, 0)),
        ],
        out_specs=(pl.BlockSpec((1, P, TM // CH, CH), lambda b, m: (b, 0, m, 0)),
                   pl.BlockSpec((1, P, TM // CH), lambda b, m: (b, 0, m))),
    )(q, em_K.transpose(0, 2, 1), em_Sp)

    cidx = pl.pallas_call(
        _chunk_topk_body,
        out_shape=jax.ShapeDtypeStruct((B, P, K_RET), jnp.int32),
        grid=(B,),
        in_specs=[pl.BlockSpec((1, P, G), lambda b: (b, 0, 0))],
        out_specs=pl.BlockSpec((1, P, K_RET), lambda b: (b, 0, 0)),
    )(cmax)

    cid_flat = cidx.reshape(B * P * K_RET)
    cands = _gather_chunks_sc(scores.reshape(B * P * G, CH), cid_flat)

    vals, jidx = pl.pallas_call(
        _cand_topk_body,
        out_shape=(jax.ShapeDtypeStruct((B, P, K_RET), jnp.float32),
                   jax.ShapeDtypeStruct((B, P, K_RET), jnp.int32)),
        grid=(B, 2),
        in_specs=[pl.BlockSpec((1, P // 2, K_RET, CH),
                               lambda b, h: (b, h, 0, 0)),
                  pl.BlockSpec((1, P // 2, K_RET), lambda b, h: (b, h, 0))],
        out_specs=(pl.BlockSpec((1, P // 2, K_RET), lambda b, h: (b, h, 0)),
                   pl.BlockSpec((1, P // 2, K_RET), lambda b, h: (b, h, 0))),
    )(cands.reshape(B, P, K_RET, CH), cidx)

    v_pairs = pl.pallas_call(
        _repack_v_body,
        out_shape=jax.ShapeDtypeStruct((B, MV2, 2 * D_EM), jnp.float32),
        grid=(B, GV),
        in_specs=[pl.BlockSpec((1, D_EM, TV), lambda b, m: (b, 0, m))],
        out_specs=pl.BlockSpec((1, TV // 2, 2 * D_EM), lambda b, m: (b, m, 0)),
    )(em_V.transpose(0, 2, 1))

    v_top = _gather_v_sc(v_pairs.reshape(B * MV2, 2 * D_EM),
                         jidx.reshape(B * P * K_RET))

    y = pl.pallas_call(
        _combine_body,
        out_shape=jax.ShapeDtypeStruct((B, P, D), jnp.float32),
        grid=(B,),
        in_specs=[
            pl.BlockSpec((1, P, D_EM), lambda b: (b, 0, 0)),
            pl.BlockSpec((1, P, K_RET, 2 * D_EM), lambda b: (b, 0, 0, 0)),
            pl.BlockSpec((1, P, K_RET), lambda b: (b, 0, 0)),
            pl.BlockSpec((1, P, K_RET), lambda b: (b, 0, 0)),
            pl.BlockSpec((D, D_EM), lambda b: (0, 0)),
            pl.BlockSpec((D,), lambda b: (0,)),
        ],
        out_specs=pl.BlockSpec((1, P, D), lambda b: (b, 0, 0)),
    )(q_cross, v_top.reshape(B, P, K_RET, 2 * D_EM), vals, jidx,
      W_o_cross, b_o_cross)

    return y


# speedup vs baseline: 1.0214x; 1.0214x over previous
import functools

import jax
import jax.numpy as jnp
from jax.experimental import pallas as pl
from jax.experimental.pallas import tpu as pltpu
from jax.experimental.pallas import tpu_sc as plsc

D_EM = 64
K_RET = 32
CH = 128
G = 896
MP = G * CH
TM = 16384
NEG = -1.0e30
NEGBIG = -3.0e38


def _q_body(x_ref, y_ref, wqem_ref, bqem_ref, wqc_ref, bqc_ref, q_ref, qc_ref):
    x = x_ref[0]
    y = y_ref[0]
    w = wqem_ref[...]
    q = (jax.lax.dot_general(x, w[:, :1024], (((1,), (1,)), ((), ())),
                             preferred_element_type=jnp.float32)
         + jax.lax.dot_general(y, w[:, 1024:], (((1,), (1,)), ((), ())),
                               preferred_element_type=jnp.float32)
         + bqem_ref[...][None, :])
    q = q * jax.lax.rsqrt(jnp.sum(q * q, axis=1, keepdims=True) + 1e-12)
    q_ref[0] = q
    qc_ref[0] = (jax.lax.dot_general(x, wqc_ref[...], (((1,), (1,)), ((), ())),
                                     preferred_element_type=jnp.float32)
                 + bqc_ref[...][None, :])


def _scores_body(q_ref, k_ref, s_ref, scores_ref, cmax_ref):
    m = pl.program_id(1)
    s = jax.lax.dot_general(q_ref[0], k_ref[0], (((1,), (0,)), ((), ())),
                            preferred_element_type=jnp.float32)
    col = m * TM + jax.lax.broadcasted_iota(jnp.int32, s.shape, 1)
    active = (s_ref[0, 0][None, :] > 0.0) & (col < 100000)
    s = jnp.where(active, s, NEG).reshape(s.shape[0], TM // CH, CH)
    scores_ref[0] = s
    cmax_ref[0] = jnp.max(s, axis=2)


def _topk_extract(c, pay, k):
    iota_k = jax.lax.broadcasted_iota(jnp.int32, (c.shape[0], k), 1)

    def body(i, carry):
        c, m, vals, idx = carry
        j = jnp.min(jnp.where(c >= m, pay, jnp.int32(2**31 - 1)),
                    axis=1, keepdims=True)
        c = jnp.where(pay == j, NEGBIG, c)
        sel = iota_k == i
        vals = jnp.where(sel, m, vals)
        idx = jnp.where(sel, j, idx)
        return c, jnp.max(c, axis=1, keepdims=True), vals, idx

    vals0 = jnp.full((c.shape[0], k), NEGBIG, jnp.float32)
    idx0 = jnp.zeros((c.shape[0], k), jnp.int32)
    m0 = jnp.max(c, axis=1, keepdims=True)
    _, _, vals, idx = jax.lax.fori_loop(0, k, body, (c, m0, vals0, idx0))
    return vals, idx


def _chunk_topk_body(cmax_ref, cidx_ref):
    c = cmax_ref[0]
    _, idx = _topk_extract(c, jax.lax.broadcasted_iota(jnp.int32, c.shape, 1),
                           K_RET)
    cidx_ref[0] = idx


def _cand_topk_body(cand_ref, cidx_ref, vals_ref, jidx_ref):
    c0 = cand_ref[0]
    p = c0.shape[0]
    gidx = (cidx_ref[0][:, :, None] * CH
            + jax.lax.broadcasted_iota(jnp.int32, (p, K_RET, CH), 2))
    iota_k = jax.lax.broadcasted_iota(jnp.int32, (p, K_RET), 1)

    def body(i, carry):
        c, m, vals, idx = carry
        j = jnp.min(jnp.where(c >= m[:, :, None], gidx,
                              jnp.int32(2**31 - 1)),
                    axis=(1, 2), keepdims=True)[:, :, 0]
        c = jnp.where(gidx == j[:, :, None], NEGBIG, c)
        sel = iota_k == i
        vals = jnp.where(sel, m, vals)
        idx = jnp.where(sel, j, idx)
        m = jnp.max(c, axis=(1, 2), keepdims=True)[:, :, 0]
        return c, m, vals, idx

    vals0 = jnp.full((p, K_RET), NEGBIG, jnp.float32)
    idx0 = jnp.zeros((p, K_RET), jnp.int32)
    m0 = jnp.max(c0, axis=(1, 2), keepdims=True)[:, :, 0]
    _, _, vals, idx = jax.lax.fori_loop(0, K_RET, body, (c0, m0, vals0, idx0))
    vals_ref[0] = vals
    jidx_ref[0] = idx


TV = 8192
GV = 13
MV2 = GV * TV // 2
LOG2TV = 13


def _repack_v_body(vt_ref, out_ref):
    v = jnp.swapaxes(vt_ref[0], 0, 1)
    out_ref[0] = jnp.concatenate([v[:TV // 2], v[TV // 2:]], axis=1)


def _combine_body(qc_ref, v_ref, vals_ref, jid_ref, wo_ref, bo_ref, y_ref):
    qc = qc_ref[0]
    v2 = v_ref[0]
    vals = vals_ref[0]
    parity = jnp.bitwise_and(
        jax.lax.shift_right_logical(jid_ref[0], LOG2TV - 1), 1)[:, :, None]
    v = jnp.where(parity == 1, v2[:, :, D_EM:], v2[:, :, :D_EM])
    t = jnp.sum(qc[:, None, :] * v, axis=2) * (D_EM ** -0.5)
    masked = vals < -1.0e29
    logits = jnp.where(masked, NEGBIG, t + vals)
    mx = jnp.max(logits, axis=1, keepdims=True)
    e = jnp.where(masked, 0.0, jnp.exp(logits - mx))
    ssum = jnp.sum(e, axis=1, keepdims=True)
    attn = e / jnp.maximum(ssum, 1e-30)
    out = jnp.sum(attn[:, :, None] * v, axis=1)
    y_ref[0] = (jax.lax.dot_general(out, wo_ref[...], (((1,), (1,)), ((), ())),
                                    preferred_element_type=jnp.float32)
                + bo_ref[...][None, :])


def _gather_chunks_sc(scores_flat, cid_flat):
    mesh = plsc.VectorSubcoreMesh(core_axis_name="c", subcore_axis_name="s")

    @functools.partial(
        pl.kernel, mesh=mesh,
        out_type=jax.ShapeDtypeStruct((16384, CH), jnp.float32),
        scratch_types=[
            pltpu.VMEM((512,), jnp.int32),
            pltpu.VMEM((4, 128), jnp.int32),
            pltpu.VMEM((512, CH), jnp.float32),
            pltpu.SemaphoreType.DMA,
        ],
    )
    def k(tab, cid, out, cid_v, idx_v, rows_v, sem):
        wid = jax.lax.axis_index("s") * 2 + jax.lax.axis_index("c")
        g0 = wid * 16
        pltpu.sync_copy(cid.at[pl.ds(g0 * K_RET, 512)], cid_v)
        for i in range(32):
            v = cid_v[pl.ds(i * 16, 16)]
            idx_v[i // 8, pl.ds((i % 8) * 16, 16)] = v + (g0 + i // 2) * G
        copies = [
            pltpu.async_copy(tab.at[idx_v.at[j]],
                             rows_v.at[pl.ds(j * 128, 128)], sem)
            for j in range(4)
        ]
        for c in copies:
            c.wait()
        pltpu.sync_copy(rows_v, out.at[pl.ds(g0 * K_RET, 512)])

    return k(scores_flat, cid_flat)


def _gather_v_sc(v_pairs, jid_flat):
    mesh = plsc.VectorSubcoreMesh(core_axis_name="c", subcore_axis_name="s")

    @functools.partial(
        pl.kernel, mesh=mesh,
        out_type=jax.ShapeDtypeStruct((16384, 2 * D_EM), jnp.float32),
        scratch_types=[
            pltpu.VMEM((512,), jnp.int32),
            pltpu.VMEM((4, 128), jnp.int32),
            pltpu.VMEM((512, 2 * D_EM), jnp.float32),
            pltpu.SemaphoreType.DMA,
        ],
    )
    def k(tab, jid, out, jid_v, idx_v, rows_v, sem):
        wid = jax.lax.axis_index("s") * 2 + jax.lax.axis_index("c")
        g0 = wid * 16
        pltpu.sync_copy(jid.at[pl.ds(g0 * K_RET, 512)], jid_v)
        for i in range(32):
            jv = jnp.minimum(jid_v[pl.ds(i * 16, 16)], 100000 - 1)
            b = jax.lax.div(g0 + i // 2, 128)
            t = jax.lax.shift_right_logical(jv, LOG2TV)
            rr = jnp.bitwise_and(jv, TV // 2 - 1)
            idx_v[i // 8, pl.ds((i % 8) * 16, 16)] = (
                b * MV2 + t * (TV // 2) + rr)
        copies = [
            pltpu.async_copy(tab.at[idx_v.at[j]],
                             rows_v.at[pl.ds(j * 128, 128)], sem)
            for j in range(4)
        ]
        for c in copies:
            c.wait()
        pltpu.sync_copy(rows_v, out.at[pl.ds(g0 * K_RET, 512)])

    return k(v_pairs, jid_flat)


def kernel(x_all, y_wm_all, em_K, em_V, em_S, W_q_em, b_q_em, W_q_cross,
           b_q_cross, W_o_cross, b_o_cross):
    B, P, D = x_all.shape
    M = em_K.shape[1]

    q, q_cross = pl.pallas_call(
        _q_body,
        out_shape=(jax.ShapeDtypeStruct((B, P, D_EM), jnp.float32),
                   jax.ShapeDtypeStruct((B, P, D_EM), jnp.float32)),
        grid=(B,),
        in_specs=[
            pl.BlockSpec((1, P, D), lambda b: (b, 0, 0)),
            pl.BlockSpec((1, P, D), lambda b: (b, 0, 0)),
            pl.BlockSpec((D_EM, 2 * D), lambda b: (0, 0)),
            pl.BlockSpec((D_EM,), lambda b: (0,)),
            pl.BlockSpec((D_EM, D), lambda b: (0, 0)),
            pl.BlockSpec((D_EM,), lambda b: (0,)),
        ],
        out_specs=(pl.BlockSpec((1, P, D_EM), lambda b: (b, 0, 0)),
                   pl.BlockSpec((1, P, D_EM), lambda b: (b, 0, 0))),
    )(x_all, y_wm_all, W_q_em, b_q_em, W_q_cross, b_q_cross)

    em_Sp = jnp.pad(em_S, ((0, 0), (0, MP - M))).reshape(B * (MP // TM), 1, TM)

    scores, cmax = pl.pallas_call(
        _scores_body,
        out_shape=(jax.ShapeDtypeStruct((B, P, G, CH), jnp.float32),
                   jax.ShapeDtypeStruct((B, P, G), jnp.float32)),
        grid=(B, MP // TM),
        in_specs=[
            pl.BlockSpec((1, P, D_EM), lambda b, m: (b, 0, 0)),
            pl.BlockSpec((1, D_EM, TM), lambda b, m: (b, 0, m)),
            pl.BlockSpec((1, 1, TM), lambda b, m: (b * (MP // TM) + m, 0, 0)),
        ],
        out_specs=(pl.BlockSpec((1, P, TM // CH, CH), lambda b, m: (b, 0, m, 0)),
                   pl.BlockSpec((1, P, TM // CH), lambda b, m: (b, 0, m))),
    )(q, em_K.transpose(0, 2, 1), em_Sp)

    cidx = pl.pallas_call(
        _chunk_topk_body,
        out_shape=jax.ShapeDtypeStruct((B, P, K_RET), jnp.int32),
        grid=(B,),
        in_specs=[pl.BlockSpec((1, P, G), lambda b: (b, 0, 0))],
        out_specs=pl.BlockSpec((1, P, K_RET), lambda b: (b, 0, 0)),
    )(cmax)

    cid_flat = cidx.reshape(B * P * K_RET)
    cands = _gather_chunks_sc(scores.reshape(B * P * G, CH), cid_flat)

    vals, jidx = pl.pallas_call(
        _cand_topk_body,
        out_shape=(jax.ShapeDtypeStruct((B, P, K_RET), jnp.float32),
                   jax.ShapeDtypeStruct((B, P, K_RET), jnp.int32)),
        grid=(B,),
        in_specs=[pl.BlockSpec((1, P, K_RET, CH), lambda b: (b, 0, 0, 0)),
                  pl.BlockSpec((1, P, K_RET), lambda b: (b, 0, 0))],
        out_specs=(pl.BlockSpec((1, P, K_RET), lambda b: (b, 0, 0)),
                   pl.BlockSpec((1, P, K_RET), lambda b: (b, 0, 0))),
    )(cands.reshape(B, P, K_RET, CH), cidx)

    v_pairs = pl.pallas_call(
        _repack_v_body,
        out_shape=jax.ShapeDtypeStruct((B, MV2, 2 * D_EM), jnp.float32),
        grid=(B, GV),
        in_specs=[pl.BlockSpec((1, D_EM, TV), lambda b, m: (b, 0, m))],
        out_specs=pl.BlockSpec((1, TV // 2, 2 * D_EM), lambda b, m: (b, m, 0)),
    )(em_V.transpose(0, 2, 1))

    v_top = _gather_v_sc(v_pairs.reshape(B * MV2, 2 * D_EM),
                         jidx.reshape(B * P * K_RET))

    y = pl.pallas_call(
        _combine_body,
        out_shape=jax.ShapeDtypeStruct((B, P, D), jnp.float32),
        grid=(B,),
        in_specs=[
            pl.BlockSpec((1, P, D_EM), lambda b: (b, 0, 0)),
            pl.BlockSpec((1, P, K_RET, 2 * D_EM), lambda b: (b, 0, 0, 0)),
            pl.BlockSpec((1, P, K_RET), lambda b: (b, 0, 0)),
            pl.BlockSpec((1, P, K_RET), lambda b: (b, 0, 0)),
            pl.BlockSpec((D, D_EM), lambda b: (0, 0)),
            pl.BlockSpec((D,), lambda b: (0,)),
        ],
        out_specs=pl.BlockSpec((1, P, D), lambda b: (b, 0, 0)),
    )(q_cross, v_top.reshape(B, P, K_RET, 2 * D_EM), vals, jidx,
      W_o_cross, b_o_cross)

    return y
